# baseline (speedup 1.0000x reference)
import functools

import jax
import jax.numpy as jnp
from jax import lax
from jax.experimental import pallas as pl
from jax.experimental.pallas import tpu as pltpu
from jax.experimental.pallas import tpu_sc as plsc

LATENT = 64
KCODE = 1024
N_TOK = 4 * 28 * 28
N_PAD = 3200
BLK = 128
ENC_LAST = 384


def _conv(x, w, stride, pad):
    return jax.lax.conv_general_dilated(
        x, w, window_strides=(stride, stride),
        padding=[(pad, pad), (pad, pad)],
        dimension_numbers=('NCHW', 'OIHW', 'NCHW'))


def _convT(x, w):
    return jax.lax.conv_transpose(
        x, w, strides=(2, 2), padding='SAME',
        dimension_numbers=('NCHW', 'HWIO', 'NCHW'))


HW = 784


def _quant_body(z_ref, cb_ref, idx_ref):
    z = z_ref[0]
    cb = cb_ref[...]
    zc = lax.dot_general(cb, z, (((1,), (0,)), ((), ())),
                         preferred_element_type=jnp.float32)
    zn = jnp.sum(z * z, axis=0, keepdims=True)
    cn = jnp.sum(cb * cb, axis=1, keepdims=True)
    d = jnp.sqrt(jnp.maximum(zn - 2.0 * zc + cn, 0.0))
    dmin = jnp.min(d, axis=0, keepdims=True)
    jidx = lax.broadcasted_iota(jnp.int32, d.shape, 0)
    idx_ref[0] = jnp.min(jnp.where(d <= dmin, jidx, KCODE),
                         axis=0, keepdims=True)


def _quantize_tc(z_cm, codebook):
    return pl.pallas_call(
        _quant_body,
        grid=(4,),
        in_specs=[
            pl.BlockSpec((1, LATENT, HW), lambda n: (n, 0, 0)),
            pl.BlockSpec((KCODE, LATENT), lambda n: (0, 0)),
        ],
        out_specs=pl.BlockSpec((1, 1, HW), lambda n: (n, 0, 0)),
        out_shape=jax.ShapeDtypeStruct((4, 1, HW), jnp.int32),
    )(z_cm, codebook)


_NC = 2
_NS = 16
_NW_USED = 28
_ROWS = 112


_CB_DUP = 128


def _sc_gather_body(cb_hbm, idx_hbm, out_hbm, idx_v, rows_v, sem):
    wid = lax.axis_index("s") * _NC + lax.axis_index("c")

    @pl.when(wid < _NW_USED)
    def _():
        base = wid * _ROWS
        pltpu.sync_copy(idx_hbm.at[pl.ds(base, _ROWS)], idx_v)
        pltpu.async_copy(cb_hbm.at[idx_v], rows_v, sem).wait()
        pltpu.sync_copy(rows_v, out_hbm.at[pl.ds(base, _ROWS)])


@functools.cache
def _sc_gather_kernel():
    return pl.kernel(
        _sc_gather_body,
        mesh=plsc.VectorSubcoreMesh(core_axis_name="c", subcore_axis_name="s"),
        out_type=jax.ShapeDtypeStruct((N_TOK, _CB_DUP), jnp.float32),
        scratch_types=[
            pltpu.VMEM((_ROWS,), jnp.int32),
            pltpu.VMEM((_ROWS, _CB_DUP), jnp.float32),
            pltpu.SemaphoreType.DMA,
        ],
    )


def kernel(x, enc_w0, enc_b0, enc_w1, enc_b1, enc_w2, enc_b2,
           proj_w, proj_b, dec_w0, dec_b0, dec_w1, dec_b1, dec_w2, dec_b2,
           out_w, out_b, codebook):
    h = x
    for w, b in ((enc_w0, enc_b0), (enc_w1, enc_b1), (enc_w2, enc_b2)):
        h = jax.nn.relu(_conv(h, w, 2, 1) + b[None, :, None, None])
    b_ = h.shape[0]

    z = _conv(h, proj_w, 1, 0) + proj_b[None, :, None, None]
    z_cm = z.reshape(b_, LATENT, HW)

    idx_cm = _quantize_tc(z_cm, codebook)
    idx = idx_cm.reshape(N_TOK)

    cb_dup = jnp.concatenate([codebook, codebook], axis=1)
    z_q_flat = _sc_gather_kernel()(cb_dup, idx)[:, :LATENT]

    hh = ww = 28
    z_q = jnp.transpose(z_q_flat.reshape(b_, hh, ww, LATENT), (0, 3, 1, 2))

    z_q_st = z + lax.stop_gradient(z_q - z)

    g = z_q_st
    for w, b in ((dec_w0, dec_b0), (dec_w1, dec_b1), (dec_w2, dec_b2)):
        g = jax.nn.relu(_convT(g, w) + b[None, :, None, None])
    out = _conv(g, out_w, 1, 1) + out_b[None, :, None, None]
    return (out, z, z_q)


# speedup vs baseline: 1.0266x; 1.0266x over previous
import functools

import jax
import jax.numpy as jnp
from jax import lax
from jax.experimental import pallas as pl
from jax.experimental.pallas import tpu as pltpu
from jax.experimental.pallas import tpu_sc as plsc

LATENT = 64
KCODE = 1024
N_TOK = 4 * 28 * 28
N_PAD = 3200
BLK = 128
ENC_LAST = 384


def _conv(x, w, stride, pad):
    return jax.lax.conv_general_dilated(
        x, w, window_strides=(stride, stride),
        padding=[(pad, pad), (pad, pad)],
        dimension_numbers=('NCHW', 'OIHW', 'NCHW'))


def _convT(x, w):
    return jax.lax.conv_transpose(
        x, w, strides=(2, 2), padding='SAME',
        dimension_numbers=('NCHW', 'HWIO', 'NCHW'))


HW = 784


def _quant_body(z_ref, cb_ref, idx_ref):
    z = z_ref[0]
    cb = cb_ref[...]
    zc = lax.dot_general(cb, z, (((1,), (0,)), ((), ())),
                         preferred_element_type=jnp.float32)
    zn = jnp.sum(z * z, axis=0, keepdims=True)
    cn = jnp.sum(cb * cb, axis=1, keepdims=True)
    d = jnp.sqrt(jnp.maximum(zn - 2.0 * zc + cn, 0.0))
    dmin = jnp.min(d, axis=0, keepdims=True)
    jidx = lax.broadcasted_iota(jnp.int32, d.shape, 0)
    idx_ref[0] = jnp.min(jnp.where(d <= dmin, jidx, KCODE),
                         axis=0, keepdims=True)


def _quantize_tc(z_cm, codebook):
    return pl.pallas_call(
        _quant_body,
        grid=(4,),
        in_specs=[
            pl.BlockSpec((1, LATENT, HW), lambda n: (n, 0, 0)),
            pl.BlockSpec((KCODE, LATENT), lambda n: (0, 0)),
        ],
        out_specs=pl.BlockSpec((1, 1, HW), lambda n: (n, 0, 0)),
        out_shape=jax.ShapeDtypeStruct((4, 1, HW), jnp.int32),
    )(z_cm, codebook)


_NC = 2
_NS = 16
_NW_USED = 28
_ROWS = 112


_CB_DUP = 128


def _sc_gather_body(cb_hbm, idx_hbm, out_hbm, idx_v, rows_v, sem):
    wid = lax.axis_index("s") * _NC + lax.axis_index("c")

    @pl.when(wid < _NW_USED)
    def _():
        base = wid * _ROWS
        pltpu.sync_copy(idx_hbm.at[pl.ds(base, _ROWS)], idx_v)
        pltpu.async_copy(cb_hbm.at[idx_v], rows_v, sem).wait()
        pltpu.sync_copy(rows_v, out_hbm.at[pl.ds(base, _ROWS)])


@functools.cache
def _sc_gather_kernel():
    return pl.kernel(
        _sc_gather_body,
        mesh=plsc.VectorSubcoreMesh(core_axis_name="c", subcore_axis_name="s"),
        compiler_params=pltpu.CompilerParams(use_tc_tiling_on_sc=False),
        out_type=jax.ShapeDtypeStruct((N_TOK, LATENT), jnp.float32),
        scratch_types=[
            pltpu.VMEM((_ROWS,), jnp.int32),
            pltpu.VMEM((_ROWS, LATENT), jnp.float32),
            pltpu.SemaphoreType.DMA,
        ],
    )


def kernel(x, enc_w0, enc_b0, enc_w1, enc_b1, enc_w2, enc_b2,
           proj_w, proj_b, dec_w0, dec_b0, dec_w1, dec_b1, dec_w2, dec_b2,
           out_w, out_b, codebook):
    h = x
    for w, b in ((enc_w0, enc_b0), (enc_w1, enc_b1), (enc_w2, enc_b2)):
        h = jax.nn.relu(_conv(h, w, 2, 1) + b[None, :, None, None])
    b_ = h.shape[0]

    z = _conv(h, proj_w, 1, 0) + proj_b[None, :, None, None]
    z_cm = z.reshape(b_, LATENT, HW)

    idx_cm = _quantize_tc(z_cm, codebook)
    idx = idx_cm.reshape(N_TOK)

    z_q_flat = _sc_gather_kernel()(codebook, idx)

    hh = ww = 28
    z_q = jnp.transpose(z_q_flat.reshape(b_, hh, ww, LATENT), (0, 3, 1, 2))

    z_q_st = z + lax.stop_gradient(z_q - z)

    g = z_q_st
    for w, b in ((dec_w0, dec_b0), (dec_w1, dec_b1), (dec_w2, dec_b2)):
        g = jax.nn.relu(_convT(g, w) + b[None, :, None, None])
    out = _conv(g, out_w, 1, 1) + out_b[None, :, None, None]
    return (out, z, z_q)


# speedup vs baseline: 1.0270x; 1.0003x over previous
import functools

import jax
import jax.numpy as jnp
from jax import lax
from jax.experimental import pallas as pl
from jax.experimental.pallas import tpu as pltpu
from jax.experimental.pallas import tpu_sc as plsc

LATENT = 64
KCODE = 1024
N_TOK = 4 * 28 * 28
ENC_LAST = 384


def _conv(x, w, stride, pad):
    return jax.lax.conv_general_dilated(
        x, w, window_strides=(stride, stride),
        padding=[(pad, pad), (pad, pad)],
        dimension_numbers=('NCHW', 'OIHW', 'NCHW'))


def _convT(x, w):
    return jax.lax.conv_transpose(
        x, w, strides=(2, 2), padding='SAME',
        dimension_numbers=('NCHW', 'HWIO', 'NCHW'))


HW = 784


def _quant_body(z_ref, cb_ref, idx_ref):
    z = z_ref[0]
    cb = cb_ref[...]
    zc = lax.dot_general(cb, z, (((1,), (0,)), ((), ())),
                         preferred_element_type=jnp.float32)
    zn = jnp.sum(z * z, axis=0, keepdims=True)
    cn = jnp.sum(cb * cb, axis=1, keepdims=True)
    d = jnp.sqrt(jnp.maximum(zn - 2.0 * zc + cn, 0.0))
    dmin = jnp.min(d, axis=0, keepdims=True)
    jidx = lax.broadcasted_iota(jnp.int32, d.shape, 0)
    idx_ref[0] = jnp.min(jnp.where(d <= dmin, jidx, KCODE),
                         axis=0, keepdims=True)


def _quantize_tc(z_cm, codebook):
    return pl.pallas_call(
        _quant_body,
        grid=(4,),
        in_specs=[
            pl.BlockSpec((1, LATENT, HW), lambda n: (n, 0, 0)),
            pl.BlockSpec((KCODE, LATENT), lambda n: (0, 0)),
        ],
        out_specs=pl.BlockSpec((1, 1, HW), lambda n: (n, 0, 0)),
        out_shape=jax.ShapeDtypeStruct((4, 1, HW), jnp.int32),
    )(z_cm, codebook)


_NC = 2
_NS = 16
_NW_USED = 28
_ROWS = 112


def _sc_gather_body(cb_hbm, idx_hbm, out_hbm, idx_v, rows_v, sem):
    wid = lax.axis_index("s") * _NC + lax.axis_index("c")

    @pl.when(wid < _NW_USED)
    def _():
        base = wid * _ROWS
        pltpu.sync_copy(idx_hbm.at[pl.ds(base, _ROWS)], idx_v)
        pltpu.async_copy(cb_hbm.at[idx_v], rows_v, sem).wait()
        pltpu.sync_copy(rows_v, out_hbm.at[pl.ds(base, _ROWS)])


@functools.cache
def _sc_gather_kernel():
    return pl.kernel(
        _sc_gather_body,
        mesh=plsc.VectorSubcoreMesh(core_axis_name="c", subcore_axis_name="s"),
        compiler_params=pltpu.CompilerParams(use_tc_tiling_on_sc=False),
        out_type=jax.ShapeDtypeStruct((N_TOK, LATENT), jnp.float32),
        scratch_types=[
            pltpu.VMEM((_ROWS,), jnp.int32),
            pltpu.VMEM((_ROWS, LATENT), jnp.float32),
            pltpu.SemaphoreType.DMA,
        ],
    )


def kernel(x, enc_w0, enc_b0, enc_w1, enc_b1, enc_w2, enc_b2,
           proj_w, proj_b, dec_w0, dec_b0, dec_w1, dec_b1, dec_w2, dec_b2,
           out_w, out_b, codebook):
    h = x
    for w, b in ((enc_w0, enc_b0), (enc_w1, enc_b1), (enc_w2, enc_b2)):
        h = jax.nn.relu(_conv(h, w, 2, 1) + b[None, :, None, None])
    b_ = h.shape[0]

    z = _conv(h, proj_w, 1, 0) + proj_b[None, :, None, None]
    z_cm = z.reshape(b_, LATENT, HW)

    idx_cm = _quantize_tc(z_cm, codebook)
    idx = idx_cm.reshape(N_TOK)

    z_q_flat = _sc_gather_kernel()(codebook, idx)

    hh = ww = 28
    z_q = jnp.transpose(z_q_flat.reshape(b_, hh, ww, LATENT), (0, 3, 1, 2))

    z_q_st = z + lax.stop_gradient(z_q - z)

    g = z_q_st
    for w, b in ((dec_w0, dec_b0), (dec_w1, dec_b1), (dec_w2, dec_b2)):
        g = jax.nn.relu(_convT(g, w) + b[None, :, None, None])
    out = _conv(g, out_w, 1, 1) + out_b[None, :, None, None]
    return (out, z, z_q)
